# baseline (speedup 1.0000x reference)
import functools

import jax
import jax.numpy as jnp
from jax import lax
from jax.experimental import pallas as pl
from jax.experimental.pallas import tpu as pltpu
from jax.experimental.pallas import tpu_sc as plsc

B = 16384
D = 5120
V = 9
NC = 2
NS = 16
NW = NC * NS
BPW = B // NW
G = 8
NGRP = BPW // G
COLS_PER_STEP = 2
NSTEP = D // COLS_PER_STEP
U = 16
L = 16


def _embed_lookup(aspect_ratio_ids, embedding_table):
    mesh = plsc.VectorSubcoreMesh(core_axis_name="c", subcore_axis_name="s")

    @functools.partial(
        pl.kernel,
        mesh=mesh,
        out_type=jax.ShapeDtypeStruct((B, D), jnp.float32),
        compiler_params=pltpu.CompilerParams(
            needs_layout_passes=False, use_tc_tiling_on_sc=False
        ),
        scratch_types=[
            pltpu.VMEM((BPW,), jnp.int32),
            pltpu.VMEM((V, D), jnp.float32),
            pltpu.VMEM((2, G, D), jnp.float32),
            pltpu.SemaphoreType.DMA,
            pltpu.SemaphoreType.DMA,
        ],
    )
    def k(idx_hbm, table_hbm, out_hbm, idx_v, table_v, rows_v, sem0, sem1):
        cid = lax.axis_index("c")
        sid = lax.axis_index("s")
        wid = sid * NC + cid
        base = wid * BPW

        pltpu.sync_copy(idx_hbm.at[pl.ds(base, BPW)], idx_v)
        pltpu.sync_copy(table_hbm, table_v)

        lane = lax.iota(jnp.int32, L)
        lane8 = lane & 7
        colsel = lane >> 3
        sems = (sem0, sem1)

        def out_copy(g, b):
            return pltpu.make_async_copy(
                rows_v.at[b], out_hbm.at[pl.ds(base + g * G, G)], sems[b]
            )

        def compute_group(g, b):
            rows8 = plsc.load_gather(idx_v, [g * G + lane8])
            buf = rows_v.at[b]

            def step(i, col_v):
                for u in range(U):
                    col = col_v + (COLS_PER_STEP * u)
                    vals = plsc.load_gather(table_v, [rows8, col])
                    plsc.store_scatter(buf, [lane8, col], vals)
                return col_v + (COLS_PER_STEP * U)

            lax.fori_loop(0, NSTEP // U, step, colsel)

        def body(i, carry):
            for b in (0, 1):
                g = i * 2 + b

                @pl.when(g >= 2)
                def _drain():
                    out_copy(g - 2, b).wait()

                compute_group(g, b)
                out_copy(g, b).start()
            return carry

        lax.fori_loop(0, NGRP // 2, body, 0)
        out_copy(NGRP - 2, 0).wait()
        out_copy(NGRP - 1, 1).wait()

    return k(aspect_ratio_ids, embedding_table)


def kernel(aspect_ratio_ids, embedding_table):
    ids = aspect_ratio_ids.astype(jnp.int32)
    return _embed_lookup(ids, embedding_table)


# speedup vs baseline: 2.1671x; 2.1671x over previous
import functools

import jax
import jax.numpy as jnp
from jax import lax
from jax.experimental import pallas as pl
from jax.experimental.pallas import tpu as pltpu
from jax.experimental.pallas import tpu_sc as plsc

B = 16384
D = 5120
V = 9
NC = 2
NS = 16
NW = NC * NS
BPW = B // NW
G = 8
NGRP = BPW // G
COLS_PER_STEP = 2
NSTEP = D // COLS_PER_STEP
U = 16
L = 16


def _embed_lookup(aspect_ratio_ids, embedding_table):
    mesh = plsc.VectorSubcoreMesh(core_axis_name="c", subcore_axis_name="s")

    @functools.partial(
        pl.kernel,
        mesh=mesh,
        out_type=jax.ShapeDtypeStruct((B, D), jnp.float32),
        compiler_params=pltpu.CompilerParams(
            needs_layout_passes=False, use_tc_tiling_on_sc=False
        ),
        scratch_types=[
            pltpu.VMEM((BPW,), jnp.int32),
            pltpu.VMEM((V, D), jnp.float32),
            pltpu.VMEM((2, G, D), jnp.float32),
            pltpu.SemaphoreType.DMA,
            pltpu.SemaphoreType.DMA,
        ],
    )
    def k(idx_hbm, table_hbm, out_hbm, idx_v, table_v, rows_v, sem0, sem1):
        cid = lax.axis_index("c")
        sid = lax.axis_index("s")
        wid = sid * NC + cid
        base = wid * BPW

        pltpu.sync_copy(idx_hbm.at[pl.ds(base, BPW)], idx_v)
        pltpu.sync_copy(table_hbm, table_v)

        lane = lax.iota(jnp.int32, L)
        lane8 = lane & 7
        colsel = lane >> 3
        sems = (sem0, sem1)

        def out_copy(g, b):
            return pltpu.make_async_copy(
                rows_v.at[b], out_hbm.at[pl.ds(base + g * G, G)], sems[b]
            )

        def compute_group(g, b):
            rows8 = plsc.load_gather(idx_v, [g * G + lane8])
            buf = rows_v.at[b]

            @plsc.parallel_loop(0, NSTEP, unroll=U)
            def _step(i):
                col = colsel + i * COLS_PER_STEP
                vals = plsc.load_gather(table_v, [rows8, col])
                plsc.store_scatter(buf, [lane8, col], vals)

        def body(i, carry):
            for b in (0, 1):
                g = i * 2 + b

                @pl.when(g >= 2)
                def _drain():
                    out_copy(g - 2, b).wait()

                compute_group(g, b)
                out_copy(g, b).start()
            return carry

        lax.fori_loop(0, NGRP // 2, body, 0)
        out_copy(NGRP - 2, 0).wait()
        out_copy(NGRP - 1, 1).wait()

    return k(aspect_ratio_ids, embedding_table)


def kernel(aspect_ratio_ids, embedding_table):
    ids = aspect_ratio_ids.astype(jnp.int32)
    return _embed_lookup(ids, embedding_table)


# speedup vs baseline: 3.3962x; 1.5671x over previous
import functools

import jax
import jax.numpy as jnp
from jax import lax
from jax.experimental import pallas as pl
from jax.experimental.pallas import tpu as pltpu
from jax.experimental.pallas import tpu_sc as plsc

B = 16384
D = 5120
V = 9
NC = 2
NS = 16
NW = NC * NS
BPW = B // NW
G = 8
NGRP = BPW // G
COLS_PER_STEP = 2
NSTEP = D // COLS_PER_STEP
U = 16
L = 16


def _embed_lookup(aspect_ratio_ids, embedding_table):
    mesh = plsc.VectorSubcoreMesh(core_axis_name="c", subcore_axis_name="s")

    @functools.partial(
        pl.kernel,
        mesh=mesh,
        out_type=jax.ShapeDtypeStruct((B, D), jnp.float32),
        compiler_params=pltpu.CompilerParams(
            needs_layout_passes=False, use_tc_tiling_on_sc=False
        ),
        scratch_types=[
            pltpu.VMEM((BPW,), jnp.int32),
            pltpu.VMEM((V, D), jnp.float32),
            pltpu.VMEM((2, G, D), jnp.float32),
            pltpu.SemaphoreType.DMA,
            pltpu.SemaphoreType.DMA,
        ],
    )
    def k(idx_hbm, table_hbm, out_hbm, idx_v, table_v, rows_v, sem0, sem1):
        cid = lax.axis_index("c")
        sid = lax.axis_index("s")
        wid = sid * NC + cid
        base = wid * BPW

        pltpu.sync_copy(idx_hbm.at[pl.ds(base, BPW)], idx_v)
        pltpu.sync_copy(table_hbm, table_v)

        lane = lax.iota(jnp.int32, L)
        sems = (sem0, sem1)

        def out_copy(g, b):
            return pltpu.make_async_copy(
                rows_v.at[b], out_hbm.at[pl.ds(base + g * G, G)], sems[b]
            )

        def compute_group(g, b):
            buf = rows_v.at[b]
            for e in range(G):
                row_s = plsc.load_gather(idx_v, [lane * 0 + (g * G + e)])
                row_buf = buf.at[e]

                @plsc.parallel_loop(0, D // L, unroll=U)
                def _step(i, row_s=row_s, row_buf=row_buf):
                    col = lane + i * L
                    vals = plsc.load_gather(table_v, [row_s, col])
                    row_buf[pl.ds(i * L, L)] = vals

        def body(i, carry):
            for b in (0, 1):
                g = i * 2 + b

                @pl.when(g >= 2)
                def _drain():
                    out_copy(g - 2, b).wait()

                compute_group(g, b)
                out_copy(g, b).start()
            return carry

        lax.fori_loop(0, NGRP // 2, body, 0)
        out_copy(NGRP - 2, 0).wait()
        out_copy(NGRP - 1, 1).wait()

    return k(aspect_ratio_ids, embedding_table)


def kernel(aspect_ratio_ids, embedding_table):
    ids = aspect_ratio_ids.astype(jnp.int32)
    return _embed_lookup(ids, embedding_table)


# speedup vs baseline: 3.4684x; 1.0213x over previous
import functools

import jax
import jax.numpy as jnp
from jax import lax
from jax.experimental import pallas as pl
from jax.experimental.pallas import tpu as pltpu
from jax.experimental.pallas import tpu_sc as plsc

B = 16384
D = 5120
V = 9
NC = 2
NS = 16
NW = NC * NS
BPW = B // NW
L = 16
NCHUNK = BPW // L


def _embed_lookup(aspect_ratio_ids, embedding_table):
    mesh = plsc.VectorSubcoreMesh(core_axis_name="c", subcore_axis_name="s")

    @functools.partial(
        pl.kernel,
        mesh=mesh,
        out_type=jax.ShapeDtypeStruct((B, D), jnp.float32),
        compiler_params=pltpu.CompilerParams(
            needs_layout_passes=False, use_tc_tiling_on_sc=False
        ),
        scratch_types=[
            pltpu.VMEM((BPW,), jnp.int32),
            pltpu.VMEM((V, D), jnp.float32),
            [pltpu.SemaphoreType.DMA] * L,
        ],
    )
    def k(idx_hbm, table_hbm, out_hbm, idx_v, table_v, sems):
        cid = lax.axis_index("c")
        sid = lax.axis_index("s")
        wid = sid * NC + cid
        base = wid * BPW

        pltpu.sync_copy(idx_hbm.at[pl.ds(base, BPW)], idx_v)
        pltpu.sync_copy(table_hbm, table_v)

        lane = lax.iota(jnp.int32, L)
        zero = lane * 0

        def row_copy(row, g, q):
            return pltpu.make_async_copy(
                table_v.at[row], out_hbm.at[base + g], sems[q]
            )

        def body(c, carry):
            chunk = idx_v[pl.ds(c * L, L)]
            for e in range(L):
                row = lax.reduce_sum(
                    lax.select(lane == e, chunk, zero), axes=(0,)
                )
                g = c * L + e

                @pl.when(c > 0)
                def _drain():
                    row_copy(0, g - L, e).wait()

                row_copy(row, g, e).start()
            return carry

        lax.fori_loop(0, NCHUNK, body, 0)
        for e in range(L):
            row_copy(0, BPW - L + e, e).wait()

    return k(aspect_ratio_ids, embedding_table)


def kernel(aspect_ratio_ids, embedding_table):
    ids = aspect_ratio_ids.astype(jnp.int32)
    return _embed_lookup(ids, embedding_table)


# speedup vs baseline: 10.9169x; 3.1475x over previous
import functools

import jax
import jax.numpy as jnp
from jax import lax
from jax.experimental import pallas as pl
from jax.experimental.pallas import tpu as pltpu
from jax.experimental.pallas import tpu_sc as plsc

B = 16384
D = 5120
V = 9
NC = 2
NS = 16
NW = NC * NS
BPW = B // NW
L = 16
NCHUNK = BPW // L


def _embed_lookup(aspect_ratio_ids, embedding_table):
    mesh = plsc.VectorSubcoreMesh(core_axis_name="c", subcore_axis_name="s")

    @functools.partial(
        pl.kernel,
        mesh=mesh,
        out_type=jax.ShapeDtypeStruct((B, D), jnp.float32),
        compiler_params=pltpu.CompilerParams(needs_layout_passes=False),
        scratch_types=[
            pltpu.VMEM((BPW,), jnp.int32),
            pltpu.VMEM((V, D), jnp.float32),
            [pltpu.SemaphoreType.DMA] * L,
        ],
    )
    def k(idx_hbm, table_hbm, out_hbm, idx_v, table_v, sems):
        cid = lax.axis_index("c")
        sid = lax.axis_index("s")
        wid = sid * NC + cid
        base = wid * BPW

        pltpu.sync_copy(idx_hbm.at[pl.ds(base, BPW)], idx_v)
        pltpu.sync_copy(table_hbm, table_v)

        lane = lax.iota(jnp.int32, L)
        zero = lane * 0

        def row_copy(row, g, q):
            return pltpu.make_async_copy(
                table_v.at[row], out_hbm.at[base + g], sems[q]
            )

        def body(c, carry):
            chunk = idx_v[pl.ds(c * L, L)]
            for e in range(L):
                row = lax.reduce_sum(
                    lax.select(lane == e, chunk, zero), axes=(0,)
                )
                g = c * L + e

                @pl.when(c > 0)
                def _drain():
                    row_copy(0, g - L, e).wait()

                row_copy(row, g, e).start()
            return carry

        lax.fori_loop(0, NCHUNK, body, 0)
        for e in range(L):
            row_copy(0, BPW - L + e, e).wait()

    return k(aspect_ratio_ids, embedding_table)


def kernel(aspect_ratio_ids, embedding_table):
    ids = aspect_ratio_ids.astype(jnp.int32)
    return _embed_lookup(ids, embedding_table)
